# Initial kernel scaffold; baseline (speedup 1.0000x reference)
import jax
import jax.numpy as jnp
from jax.experimental import pallas as pl


def kernel(feat, E1W, E1b, E2W, E2b, F1W, F1b, F2W, F2b, g0, b0, rm0, rv0, W1, b1, W2, b2, W3, b3):
    raise NotImplementedError("write your pallas kernel here")


# speedup vs baseline: 4.2175x; 4.2175x over previous
import jax
import jax.numpy as jnp
from jax.experimental import pallas as pl
from jax.experimental.pallas import tpu as pltpu

N = 512
KNB = 8
BI = 8
M = N * (N - 1) // 2
SPLIT = 152
OFF_SPLIT = SPLIT * (N - 1) - SPLIT * (SPLIT - 1) // 2
M2 = M - OFF_SPLIT


def _leaky(v):
    return jnp.where(v >= 0, v, 0.01 * v)


def _prep_kernel(x_ref, posT_ref, Wk1_ref, E1b_ref, E2WT_ref, E2b_ref,
                 Fk_ref, F1b_ref, F2WT_ref, F2b_ref,
                 Ax_ref, Ae1_ref, Ae2_ref, Bx_ref, Be1_ref, Be2_ref,
                 U_ref, V_ref):
    x = x_ref[...]
    lane = jax.lax.broadcasted_iota(jnp.int32, (N, N), 1)
    d = jnp.zeros((N, N), jnp.float32)
    for c in range(4):
        d = d + jnp.abs(x[:, c:c + 1] - posT_ref[c:c + 1, :])
    ohs = []
    big = jnp.float32(3.4e38)
    for _ in range(KNB):
        m = jnp.min(d, axis=1, keepdims=True)
        idxk = jnp.min(jnp.where(d == m, lane, N), axis=1, keepdims=True)
        sel = lane == idxk
        ohs.append(sel.astype(jnp.float32))
        d = jnp.where(sel, big, d)

    f32 = jnp.float32
    e1p = jnp.zeros((N, 32), f32)
    for k in range(KNB):
        xw = jnp.dot(x, Wk1_ref[k], preferred_element_type=f32)
        e1p = e1p + jnp.dot(ohs[k], xw, preferred_element_type=f32)
    e1 = _leaky(e1p + E1b_ref[...])
    e1 = _leaky(jnp.dot(e1, E2WT_ref[...], preferred_element_type=f32)
                + E2b_ref[...])

    e2p = jnp.zeros((N, 32), f32)
    for k in range(KNB):
        ew = jnp.dot(e1, Fk_ref[k], preferred_element_type=f32)
        e2p = e2p + jnp.dot(ohs[k], ew, preferred_element_type=f32)
    e2 = _leaky(e2p + F1b_ref[...])
    e2 = _leaky(jnp.dot(e2, F2WT_ref[...], preferred_element_type=f32)
                + F2b_ref[...])

    U_ref[...] = (jnp.dot(x, Ax_ref[...], preferred_element_type=f32)
                  + jnp.dot(e1, Ae1_ref[...], preferred_element_type=f32)
                  + jnp.dot(e2, Ae2_ref[...], preferred_element_type=f32))
    V_ref[...] = (jnp.dot(x, Bx_ref[...], preferred_element_type=f32)
                  + jnp.dot(e1, Be1_ref[...], preferred_element_type=f32)
                  + jnp.dot(e2, Be2_ref[...], preferred_element_type=f32))


def _make_pair_kernel(row0, base_off):
    def _pair_kernel(U_ref, V_ref, b1_ref, W2T_ref, b2_ref, W3T_ref, b3_ref,
                     out_ref, scr_ref):
        f32 = jnp.float32
        ib = pl.program_id(0)
        u = U_ref[...]
        v = V_ref[...]
        h = _leaky(u[:, None, :] + v[None, :, :] + b1_ref[...][None, :, :])
        h = h.reshape(BI * N, 32)
        h = _leaky(jnp.dot(h, W2T_ref[...], preferred_element_type=f32)
                   + b2_ref[...])
        y = jnp.dot(h, W3T_ref[...], preferred_element_type=f32) + b3_ref[...]
        y3 = y.reshape(BI, N, 2)
        for r in range(BI):
            i = row0 + ib * BI + r
            off = i * (N - 1) - (i * (i - 1)) // 2 - base_off
            scr_ref[0:N, :] = y3[r]
            out_ref[pl.ds(off, N), :] = scr_ref[pl.ds(i + 1, N), :]
    return _pair_kernel


def kernel(feat, E1W, E1b, E2W, E2b, F1W, F1b, F2W, F2b,
           g0, b0, rm0, rv0, W1, b1, W2, b2, W3, b3):
    f32 = jnp.float32
    x = feat[0]
    posT = x[:, :4].T
    Wk1 = E1W.reshape(32, 68, KNB).transpose(2, 1, 0)
    Fk = F1W.reshape(32, 32, KNB).transpose(2, 1, 0)
    E2WT = E2W.T
    F2WT = F2W.T
    s = g0 / jnp.sqrt(rv0 + 1e-5)
    t = b0 - rm0 * s
    W1s = W1 * s[None, :]
    b1eff = (b1 + W1 @ t).reshape(1, 32)
    Ax = W1s[:, :68].T
    Ae1 = W1s[:, 68:100].T
    Ae2 = W1s[:, 100:132].T
    Bx = W1s[:, 132:200].T
    Be1 = W1s[:, 200:232].T
    Be2 = W1s[:, 232:264].T

    full = lambda shape: pl.BlockSpec(shape, lambda: (0,) * len(shape))
    U, V = pl.pallas_call(
        _prep_kernel,
        in_specs=[full((N, 68)), full((4, N)), full((KNB, 68, 32)),
                  full((1, 32)), full((32, 32)), full((1, 32)),
                  full((KNB, 32, 32)), full((1, 32)), full((32, 32)),
                  full((1, 32)),
                  full((68, 32)), full((32, 32)), full((32, 32)),
                  full((68, 32)), full((32, 32)), full((32, 32))],
        out_specs=[full((N, 32)), full((N, 32))],
        out_shape=[jax.ShapeDtypeStruct((N, 32), f32),
                   jax.ShapeDtypeStruct((N, 32), f32)],
    )(x, posT, Wk1, E1b.reshape(1, 32), E2WT, E2b.reshape(1, 32),
      Fk, F1b.reshape(1, 32), F2WT, F2b.reshape(1, 32),
      Ax, Ae1, Ae2, Bx, Be1, Be2)

    def _pair_call(row0, nrows, base_off, valid_rows):
        out_rows = valid_rows + N
        out_rows += (-out_rows) % 8
        blk0 = row0 // BI
        return pl.pallas_call(
            _make_pair_kernel(row0, base_off),
            grid=(nrows // BI,),
            in_specs=[pl.BlockSpec((BI, 32), lambda i: (i + blk0, 0)),
                      pl.BlockSpec((N, 32), lambda i: (0, 0)),
                      pl.BlockSpec((1, 32), lambda i: (0, 0)),
                      pl.BlockSpec((32, 32), lambda i: (0, 0)),
                      pl.BlockSpec((1, 32), lambda i: (0, 0)),
                      pl.BlockSpec((32, 2), lambda i: (0, 0)),
                      pl.BlockSpec((1, 2), lambda i: (0, 0))],
            out_specs=pl.BlockSpec((out_rows, 2), lambda i: (0, 0)),
            out_shape=jax.ShapeDtypeStruct((out_rows, 2), f32),
            scratch_shapes=[pltpu.VMEM((2 * N, 2), f32)],
        )(U, V, b1eff, W2.T, b2.reshape(1, 32), W3.T, b3.reshape(1, 2))

    out1 = _pair_call(0, SPLIT, 0, OFF_SPLIT)
    out2 = _pair_call(SPLIT, N - SPLIT, OFF_SPLIT, M2)
    return jnp.concatenate([out1[:OFF_SPLIT], out2[:M2]], axis=0)


# speedup vs baseline: 4.8860x; 1.1585x over previous
import jax
import jax.numpy as jnp
from jax.experimental import pallas as pl
from jax.experimental.pallas import tpu as pltpu

N = 512
KNB = 8
BI = 16
M = N * (N - 1) // 2


def _leaky(v):
    return jnp.where(v >= 0, v, 0.01 * v)


def _prep_kernel(x_ref, posT_ref, Wk1_ref, E1b_ref, E2WT_ref, E2b_ref,
                 Fk_ref, F1b_ref, F2WT_ref, F2b_ref,
                 Ax_ref, Ae1_ref, Ae2_ref, Bx_ref, Be1_ref, Be2_ref,
                 U_ref, V_ref):
    x = x_ref[...]
    lane = jax.lax.broadcasted_iota(jnp.int32, (N, N), 1)
    d = jnp.zeros((N, N), jnp.float32)
    for c in range(4):
        d = d + jnp.abs(x[:, c:c + 1] - posT_ref[c:c + 1, :])
    ohs = []
    big = jnp.float32(3.4e38)
    for _ in range(KNB):
        m = jnp.min(d, axis=1, keepdims=True)
        idxk = jnp.min(jnp.where(d == m, lane, N), axis=1, keepdims=True)
        sel = lane == idxk
        ohs.append(sel.astype(jnp.float32))
        d = jnp.where(sel, big, d)

    f32 = jnp.float32
    e1p = jnp.zeros((N, 32), f32)
    for k in range(KNB):
        xw = jnp.dot(x, Wk1_ref[k], preferred_element_type=f32)
        e1p = e1p + jnp.dot(ohs[k], xw, preferred_element_type=f32)
    e1 = _leaky(e1p + E1b_ref[...])
    e1 = _leaky(jnp.dot(e1, E2WT_ref[...], preferred_element_type=f32)
                + E2b_ref[...])

    e2p = jnp.zeros((N, 32), f32)
    for k in range(KNB):
        ew = jnp.dot(e1, Fk_ref[k], preferred_element_type=f32)
        e2p = e2p + jnp.dot(ohs[k], ew, preferred_element_type=f32)
    e2 = _leaky(e2p + F1b_ref[...])
    e2 = _leaky(jnp.dot(e2, F2WT_ref[...], preferred_element_type=f32)
                + F2b_ref[...])

    U_ref[...] = (jnp.dot(x, Ax_ref[...], preferred_element_type=f32)
                  + jnp.dot(e1, Ae1_ref[...], preferred_element_type=f32)
                  + jnp.dot(e2, Ae2_ref[...], preferred_element_type=f32))
    V_ref[...] = (jnp.dot(x, Bx_ref[...], preferred_element_type=f32)
                  + jnp.dot(e1, Be1_ref[...], preferred_element_type=f32)
                  + jnp.dot(e2, Be2_ref[...], preferred_element_type=f32))


BHEAD = N
LMAX = BI * (N - 1) - (BI * (BI - 1)) // 2


def _pair_kernel(U_ref, V_ref, b1_ref, W2T_ref, b2_ref, W3T_ref, b3_ref,
                 out_ref, scr_ref, sem):
    f32 = jnp.float32
    t = pl.program_id(0)
    i0 = t * BI
    u = U_ref[...]
    v = V_ref[...]
    h = _leaky(u[:, None, :] + v[None, :, :] + b1_ref[...][None, :, :])
    h = h.reshape(BI * N, 32)
    h = _leaky(jnp.dot(h, W2T_ref[...], preferred_element_type=f32)
               + b2_ref[...])
    y = jnp.dot(h, W3T_ref[...], preferred_element_type=f32) + b3_ref[...]
    y3 = y.reshape(BI, N, 2)
    off_t = i0 * (N - 1) - (i0 * (i0 - 1)) // 2
    L_t = LMAX - BI * i0
    loff = 0
    for r in range(BI - 1, -1, -1):
        i = i0 + r
        loff = i * (N - 1) - (i * (i - 1)) // 2 - off_t
        scr_ref[pl.ds(BHEAD + loff - i - 1, N), :] = y3[r]
    def dma(sbase, size):
        return pltpu.make_async_copy(
            scr_ref.at[pl.ds(BHEAD + sbase, size), :],
            out_ref.at[pl.ds(off_t + sbase, size), :],
            sem)
    big = L_t >= 1024
    m = L_t // 1024
    copies = []
    for k in range(LMAX // 1024):
        copies.append((big & (k < m), dma(k * 1024, 1024)))
    copies.append((big & (L_t - m * 1024 > 0), dma(L_t - 1024, 1024)))
    for s in (512, 256, 128, 64, 32, 16, 8):
        copies.append(((~big) & ((L_t & s) != 0),
                       dma(L_t - (L_t & (2 * s - 1)), s)))
    for cond, cp in copies:
        @pl.when(cond)
        def _():
            cp.start()
    for cond, cp in copies:
        @pl.when(cond)
        def _():
            cp.wait()


def kernel(feat, E1W, E1b, E2W, E2b, F1W, F1b, F2W, F2b,
           g0, b0, rm0, rv0, W1, b1, W2, b2, W3, b3):
    f32 = jnp.float32
    x = feat[0]
    posT = x[:, :4].T
    Wk1 = E1W.reshape(32, 68, KNB).transpose(2, 1, 0)
    Fk = F1W.reshape(32, 32, KNB).transpose(2, 1, 0)
    E2WT = E2W.T
    F2WT = F2W.T
    s = g0 / jnp.sqrt(rv0 + 1e-5)
    t = b0 - rm0 * s
    W1s = W1 * s[None, :]
    b1eff = (b1 + W1 @ t).reshape(1, 32)
    Ax = W1s[:, :68].T
    Ae1 = W1s[:, 68:100].T
    Ae2 = W1s[:, 100:132].T
    Bx = W1s[:, 132:200].T
    Be1 = W1s[:, 200:232].T
    Be2 = W1s[:, 232:264].T

    full = lambda shape: pl.BlockSpec(shape, lambda: (0,) * len(shape))
    U, V = pl.pallas_call(
        _prep_kernel,
        in_specs=[full((N, 68)), full((4, N)), full((KNB, 68, 32)),
                  full((1, 32)), full((32, 32)), full((1, 32)),
                  full((KNB, 32, 32)), full((1, 32)), full((32, 32)),
                  full((1, 32)),
                  full((68, 32)), full((32, 32)), full((32, 32)),
                  full((68, 32)), full((32, 32)), full((32, 32))],
        out_specs=[full((N, 32)), full((N, 32))],
        out_shape=[jax.ShapeDtypeStruct((N, 32), f32),
                   jax.ShapeDtypeStruct((N, 32), f32)],
    )(x, posT, Wk1, E1b.reshape(1, 32), E2WT, E2b.reshape(1, 32),
      Fk, F1b.reshape(1, 32), F2WT, F2b.reshape(1, 32),
      Ax, Ae1, Ae2, Bx, Be1, Be2)

    scr_rows = BHEAD + LMAX
    scr_rows += (-scr_rows) % 8
    out = pl.pallas_call(
        _pair_kernel,
        grid=(N // BI,),
        in_specs=[pl.BlockSpec((BI, 32), lambda i: (i, 0)),
                  pl.BlockSpec((N, 32), lambda i: (0, 0)),
                  pl.BlockSpec((1, 32), lambda i: (0, 0)),
                  pl.BlockSpec((32, 32), lambda i: (0, 0)),
                  pl.BlockSpec((1, 32), lambda i: (0, 0)),
                  pl.BlockSpec((32, 2), lambda i: (0, 0)),
                  pl.BlockSpec((1, 2), lambda i: (0, 0))],
        out_specs=pl.BlockSpec(memory_space=pl.ANY),
        out_shape=jax.ShapeDtypeStruct((M, 2), f32),
        scratch_shapes=[pltpu.VMEM((scr_rows, 2), f32),
                        pltpu.SemaphoreType.DMA],
    )(U, V, b1eff, W2.T, b2.reshape(1, 32), W3.T, b3.reshape(1, 2))
    return out


# speedup vs baseline: 5.6888x; 1.1643x over previous
import jax
import jax.numpy as jnp
from jax.experimental import pallas as pl
from jax.experimental.pallas import tpu as pltpu

N = 512
KNB = 8
BI = 16
M = N * (N - 1) // 2


def _leaky(v):
    return jnp.where(v >= 0, v, 0.01 * v)


def _prep_kernel(x_ref, posT_ref, Wk1_ref, E1b_ref, E2WT_ref, E2b_ref,
                 Fk_ref, F1b_ref, F2WT_ref, F2b_ref,
                 Ax_ref, Ae1_ref, Ae2_ref, Bx_ref, Be1_ref, Be2_ref,
                 U_ref, V_ref):
    x = x_ref[...]
    lane = jax.lax.broadcasted_iota(jnp.int32, (N, N), 1)
    d = jnp.zeros((N, N), jnp.float32)
    for c in range(4):
        d = d + jnp.abs(x[:, c:c + 1] - posT_ref[c:c + 1, :])
    ohs = []
    big = jnp.float32(3.4e38)
    for _ in range(KNB):
        m = jnp.min(d, axis=1, keepdims=True)
        idxk = jnp.min(jnp.where(d == m, lane, N), axis=1, keepdims=True)
        sel = lane == idxk
        ohs.append(sel.astype(jnp.float32))
        d = jnp.where(sel, big, d)

    f32 = jnp.float32
    e1p = jnp.zeros((N, 32), f32)
    for k in range(KNB):
        xw = jnp.dot(x, Wk1_ref[k], preferred_element_type=f32)
        e1p = e1p + jnp.dot(ohs[k], xw, preferred_element_type=f32)
    e1 = _leaky(e1p + E1b_ref[...])
    e1 = _leaky(jnp.dot(e1, E2WT_ref[...], preferred_element_type=f32)
                + E2b_ref[...])

    e2p = jnp.zeros((N, 32), f32)
    for k in range(KNB):
        ew = jnp.dot(e1, Fk_ref[k], preferred_element_type=f32)
        e2p = e2p + jnp.dot(ohs[k], ew, preferred_element_type=f32)
    e2 = _leaky(e2p + F1b_ref[...])
    e2 = _leaky(jnp.dot(e2, F2WT_ref[...], preferred_element_type=f32)
                + F2b_ref[...])

    U_ref[...] = (jnp.dot(x, Ax_ref[...], preferred_element_type=f32)
                  + jnp.dot(e1, Ae1_ref[...], preferred_element_type=f32)
                  + jnp.dot(e2, Ae2_ref[...], preferred_element_type=f32))
    V_ref[...] = (jnp.dot(x, Bx_ref[...], preferred_element_type=f32)
                  + jnp.dot(e1, Be1_ref[...], preferred_element_type=f32)
                  + jnp.dot(e2, Be2_ref[...], preferred_element_type=f32))


BHEAD = N
LMAX = BI * (N - 1) - (BI * (BI - 1)) // 2


NSTEP = N // BI
NCHUNK = 4


def _pair_kernel(U_ref, V_ref, b1_ref, W2bd_ref, b2_ref, W3bd_ref, b3_ref,
                 out_ref, scr_ref, sem):
    f32 = jnp.float32
    t = pl.program_id(0)
    i0 = t * BI
    u = U_ref[...]
    v = V_ref[...]
    b1 = b1_ref[...]
    rows_per_chunk = BI // NCHUNK
    crows = rows_per_chunk * N
    hs = []
    for c in range(NCHUNK):
        uc = u[c * rows_per_chunk:(c + 1) * rows_per_chunk]
        hc = uc[:, None, :] + v[None, :, :] + b1[None, :, :]
        hs.append(hc.reshape(crows, 32))
    h = jnp.concatenate(hs, axis=1)
    h = jnp.maximum(h, 0.01 * h)
    h = jnp.dot(h, W2bd_ref[...], preferred_element_type=f32) + b2_ref[...]
    h = jnp.maximum(h, 0.01 * h)
    y4 = (jnp.dot(h, W3bd_ref[...], preferred_element_type=f32)
          + b3_ref[...])
    off_t = i0 * (N - 1) - (i0 * (i0 - 1)) // 2
    L_t = LMAX - BI * i0
    buf = t % 2

    def copies_for(tq, bufq):
        iq = tq * BI
        offq = iq * (N - 1) - (iq * (iq - 1)) // 2
        Lq = LMAX - BI * iq

        def dma(sbase, size):
            return pltpu.make_async_copy(
                scr_ref.at[bufq, pl.ds(BHEAD + sbase, size), :],
                out_ref.at[pl.ds(offq + sbase, size), :],
                sem)
        bigq = Lq >= 1024
        mq = Lq // 1024
        cps = []
        for k in range(LMAX // 1024):
            cps.append((bigq & (k < mq), dma(k * 1024, 1024)))
        cps.append((bigq & (Lq - mq * 1024 > 0), dma(Lq - 1024, 1024)))
        for s in (512, 256, 128, 64, 32, 16, 8):
            cps.append(((~bigq) & ((Lq & s) != 0),
                        dma(Lq - (Lq & (2 * s - 1)), s)))
        return cps

    for cond, cp in copies_for(t - 1, (t - 1) % 2):
        @pl.when((t >= 1) & cond)
        def _():
            cp.wait()

    for r in range(BI - 1, -1, -1):
        i = i0 + r
        loff = i * (N - 1) - (i * (i - 1)) // 2 - off_t
        c, rr = divmod(r, rows_per_chunk)
        yrow = y4[rr * N:(rr + 1) * N, 2 * c:2 * c + 2]
        scr_ref[buf, pl.ds(BHEAD + loff - i - 1, N), :] = yrow
    own = copies_for(t, buf)
    for cond, cp in own:
        @pl.when(cond)
        def _():
            cp.start()
    for cond, cp in own:
        @pl.when((t == NSTEP - 1) & cond)
        def _():
            cp.wait()


def kernel(feat, E1W, E1b, E2W, E2b, F1W, F1b, F2W, F2b,
           g0, b0, rm0, rv0, W1, b1, W2, b2, W3, b3):
    f32 = jnp.float32
    x = feat[0]
    posT = x[:, :4].T
    Wk1 = E1W.reshape(32, 68, KNB).transpose(2, 1, 0)
    Fk = F1W.reshape(32, 32, KNB).transpose(2, 1, 0)
    E2WT = E2W.T
    F2WT = F2W.T
    s = g0 / jnp.sqrt(rv0 + 1e-5)
    t = b0 - rm0 * s
    W1s = W1 * s[None, :]
    b1eff = (b1 + W1 @ t).reshape(1, 32)
    Ax = W1s[:, :68].T
    Ae1 = W1s[:, 68:100].T
    Ae2 = W1s[:, 100:132].T
    Bx = W1s[:, 132:200].T
    Be1 = W1s[:, 200:232].T
    Be2 = W1s[:, 232:264].T

    full = lambda shape: pl.BlockSpec(shape, lambda: (0,) * len(shape))
    U, V = pl.pallas_call(
        _prep_kernel,
        in_specs=[full((N, 68)), full((4, N)), full((KNB, 68, 32)),
                  full((1, 32)), full((32, 32)), full((1, 32)),
                  full((KNB, 32, 32)), full((1, 32)), full((32, 32)),
                  full((1, 32)),
                  full((68, 32)), full((32, 32)), full((32, 32)),
                  full((68, 32)), full((32, 32)), full((32, 32))],
        out_specs=[full((N, 32)), full((N, 32))],
        out_shape=[jax.ShapeDtypeStruct((N, 32), f32),
                   jax.ShapeDtypeStruct((N, 32), f32)],
    )(x, posT, Wk1, E1b.reshape(1, 32), E2WT, E2b.reshape(1, 32),
      Fk, F1b.reshape(1, 32), F2WT, F2b.reshape(1, 32),
      Ax, Ae1, Ae2, Bx, Be1, Be2)

    from jax.scipy.linalg import block_diag as _bd
    W2bd = _bd(*([W2.T] * NCHUNK))
    W3bd = _bd(*([W3.T] * NCHUNK))
    b2t = jnp.tile(b2, NCHUNK).reshape(1, 32 * NCHUNK)
    b3t = jnp.tile(b3, NCHUNK).reshape(1, 2 * NCHUNK)
    scr_rows = BHEAD + LMAX
    scr_rows += (-scr_rows) % 8
    out = pl.pallas_call(
        _pair_kernel,
        grid=(N // BI,),
        in_specs=[pl.BlockSpec((BI, 32), lambda i: (i, 0)),
                  pl.BlockSpec((N, 32), lambda i: (0, 0)),
                  pl.BlockSpec((1, 32), lambda i: (0, 0)),
                  pl.BlockSpec((128, 128), lambda i: (0, 0)),
                  pl.BlockSpec((1, 128), lambda i: (0, 0)),
                  pl.BlockSpec((128, 2 * NCHUNK), lambda i: (0, 0)),
                  pl.BlockSpec((1, 2 * NCHUNK), lambda i: (0, 0))],
        out_specs=pl.BlockSpec(memory_space=pl.ANY),
        out_shape=jax.ShapeDtypeStruct((M, 2), f32),
        scratch_shapes=[pltpu.VMEM((2, scr_rows, 2), f32),
                        pltpu.SemaphoreType.DMA],
    )(U, V, b1eff, W2bd, b2t, W3bd, b3t)
    return out
